# baseline (speedup 1.0000x reference)
import jax
import jax.numpy as jnp
from jax import lax
from jax.experimental import pallas as pl
from jax.experimental.pallas import tpu as pltpu

_MAX_PATH_DISTANCE = 5.0
_R = 40
_NBUF = 4


def _body(ev_ref, w_hbm, o_hbm, in_buf, out_buf, in_sems, out_sems):
    n_rows = w_hbm.shape[0]
    n_chunks = n_rows // _R
    s = jnp.sum(ev_ref[...]) / ev_ref.size

    def in_copy(c, slot):
        return pltpu.make_async_copy(
            w_hbm.at[pl.ds(c * _R, _R), :],
            in_buf.at[pl.ds(slot * _R, _R), :],
            in_sems.at[slot],
        )

    def out_copy(c, slot):
        return pltpu.make_async_copy(
            out_buf.at[pl.ds(slot * _R, _R), :],
            o_hbm.at[pl.ds(c * _R, _R), :],
            out_sems.at[slot],
        )

    for c in range(_NBUF):
        in_copy(c, c).start()

    def step(c, carry):
        slot = lax.rem(c, _NBUF)
        in_copy(c, slot).wait()

        @pl.when(c >= _NBUF)
        def _():
            out_copy(c - _NBUF, slot).wait()

        off = slot * _R
        v = in_buf[pl.ds(off, _R), :]
        out_buf[pl.ds(off, _R), :] = jnp.nan_to_num(
            jnp.minimum(v, jnp.float32(_MAX_PATH_DISTANCE)) * s
        )
        out_copy(c, slot).start()

        @pl.when(c + _NBUF < n_chunks)
        def _():
            in_copy(c + _NBUF, slot).start()

        return carry

    lax.fori_loop(0, n_chunks, step, 0)
    for k in range(_NBUF):
        c = n_chunks - _NBUF + k
        out_copy(c, c % _NBUF).wait()


def kernel(x, edge_attr, weights, edge_vector):
    n_rows, n_cols = weights.shape
    return pl.pallas_call(
        _body,
        in_specs=[
            pl.BlockSpec(edge_vector.shape, lambda: (0, 0)),
            pl.BlockSpec(memory_space=pltpu.MemorySpace.HBM),
        ],
        out_specs=pl.BlockSpec(memory_space=pltpu.MemorySpace.HBM),
        out_shape=jax.ShapeDtypeStruct((n_rows, n_cols), jnp.float32),
        scratch_shapes=[
            pltpu.VMEM((_NBUF * _R, n_cols), jnp.float32),
            pltpu.VMEM((_NBUF * _R, n_cols), jnp.float32),
            pltpu.SemaphoreType.DMA((_NBUF,)),
            pltpu.SemaphoreType.DMA((_NBUF,)),
        ],
    )(edge_vector, weights)


# speedup vs baseline: 1.0098x; 1.0098x over previous
import jax
import jax.numpy as jnp
from jax import lax
from jax.experimental import pallas as pl
from jax.experimental.pallas import tpu as pltpu

_MAX_PATH_DISTANCE = 5.0
_R = 200
_NBUF = 3


def _body(ev_ref, w_hbm, o_hbm, in_buf, out_buf, in_sems, out_sems):
    n_rows = w_hbm.shape[0]
    n_chunks = n_rows // _R
    s = jnp.sum(ev_ref[...]) / ev_ref.size

    def in_copy(c, slot):
        return pltpu.make_async_copy(
            w_hbm.at[pl.ds(c * _R, _R), :],
            in_buf.at[pl.ds(slot * _R, _R), :],
            in_sems.at[slot],
        )

    def out_copy(c, slot):
        return pltpu.make_async_copy(
            out_buf.at[pl.ds(slot * _R, _R), :],
            o_hbm.at[pl.ds(c * _R, _R), :],
            out_sems.at[slot],
        )

    for c in range(_NBUF):
        in_copy(c, c).start()

    def step(c, carry):
        slot = lax.rem(c, _NBUF)
        in_copy(c, slot).wait()

        @pl.when(c >= _NBUF)
        def _():
            out_copy(c - _NBUF, slot).wait()

        off = slot * _R
        v = in_buf[pl.ds(off, _R), :]
        out_buf[pl.ds(off, _R), :] = jnp.nan_to_num(
            jnp.minimum(v, jnp.float32(_MAX_PATH_DISTANCE)) * s
        )
        out_copy(c, slot).start()

        @pl.when(c + _NBUF < n_chunks)
        def _():
            in_copy(c + _NBUF, slot).start()

        return carry

    lax.fori_loop(0, n_chunks, step, 0)
    for k in range(_NBUF):
        c = n_chunks - _NBUF + k
        out_copy(c, c % _NBUF).wait()


def kernel(x, edge_attr, weights, edge_vector):
    n_rows, n_cols = weights.shape
    return pl.pallas_call(
        _body,
        in_specs=[
            pl.BlockSpec(edge_vector.shape, lambda: (0, 0)),
            pl.BlockSpec(memory_space=pltpu.MemorySpace.HBM),
        ],
        out_specs=pl.BlockSpec(memory_space=pltpu.MemorySpace.HBM),
        out_shape=jax.ShapeDtypeStruct((n_rows, n_cols), jnp.float32),
        scratch_shapes=[
            pltpu.VMEM((_NBUF * _R, n_cols), jnp.float32),
            pltpu.VMEM((_NBUF * _R, n_cols), jnp.float32),
            pltpu.SemaphoreType.DMA((_NBUF,)),
            pltpu.SemaphoreType.DMA((_NBUF,)),
        ],
    )(edge_vector, weights)


# speedup vs baseline: 1.0099x; 1.0001x over previous
import jax
import jax.numpy as jnp
from jax import lax
from jax.experimental import pallas as pl
from jax.experimental.pallas import tpu as pltpu

_MAX_PATH_DISTANCE = 5.0
_R = 200
_NIN = 4
_NOUT = 3


def _body(ev_ref, w_hbm, o_hbm, in_buf, out_buf, in_sems, out_sems):
    n_rows = w_hbm.shape[0]
    n_chunks = n_rows // _R
    s = jnp.sum(ev_ref[...]) / ev_ref.size

    def in_copy(c, slot):
        return pltpu.make_async_copy(
            w_hbm.at[pl.ds(c * _R, _R), :],
            in_buf.at[pl.ds(slot * _R, _R), :],
            in_sems.at[slot],
        )

    def out_copy(c, slot):
        return pltpu.make_async_copy(
            out_buf.at[pl.ds(slot * _R, _R), :],
            o_hbm.at[pl.ds(c * _R, _R), :],
            out_sems.at[slot],
        )

    for c in range(_NIN):
        in_copy(c, c).start()

    def step(c, carry):
        islot = lax.rem(c, _NIN)
        oslot = lax.rem(c, _NOUT)
        in_copy(c, islot).wait()

        @pl.when(c >= _NOUT)
        def _():
            out_copy(c - _NOUT, oslot).wait()

        v = in_buf[pl.ds(islot * _R, _R), :]
        out_buf[pl.ds(oslot * _R, _R), :] = jnp.nan_to_num(
            jnp.minimum(v, jnp.float32(_MAX_PATH_DISTANCE)) * s
        )
        out_copy(c, oslot).start()

        @pl.when(c + _NIN < n_chunks)
        def _():
            in_copy(c + _NIN, islot).start()

        return carry

    lax.fori_loop(0, n_chunks, step, 0)
    for k in range(_NOUT):
        c = n_chunks - _NOUT + k
        out_copy(c, c % _NOUT).wait()


def kernel(x, edge_attr, weights, edge_vector):
    n_rows, n_cols = weights.shape
    return pl.pallas_call(
        _body,
        in_specs=[
            pl.BlockSpec(edge_vector.shape, lambda: (0, 0)),
            pl.BlockSpec(memory_space=pltpu.MemorySpace.HBM),
        ],
        out_specs=pl.BlockSpec(memory_space=pltpu.MemorySpace.HBM),
        out_shape=jax.ShapeDtypeStruct((n_rows, n_cols), jnp.float32),
        scratch_shapes=[
            pltpu.VMEM((_NIN * _R, n_cols), jnp.float32),
            pltpu.VMEM((_NOUT * _R, n_cols), jnp.float32),
            pltpu.SemaphoreType.DMA((_NIN,)),
            pltpu.SemaphoreType.DMA((_NOUT,)),
        ],
    )(edge_vector, weights)
